# baseline (speedup 1.0000x reference)
import jax
import jax.numpy as jnp
import numpy as np
from jax.experimental import pallas as pl
from jax.experimental.pallas import tpu as pltpu

N = 4096
KOFF = 63
BLOCK_ROWS = 512
INT32_MIN = np.int32(-2147483648)
SIGN_LO = np.int32(0x7FFFFFFF)
ONE16 = np.int16(1)
ZERO16 = np.int16(0)


def _make_key(x, i, r):
    bits = jax.lax.bitcast_convert_type(x, jnp.int32)
    key = bits ^ (jax.lax.shift_right_arithmetic(bits, 31) & SIGN_LO)
    row = jax.lax.broadcasted_iota(jnp.int32, (r, N), 0) + i * r
    col = jax.lax.broadcasted_iota(jnp.int32, (r, N), 1)
    return key, row == col


def _block_kernel(x_ref, o_ref):
    i = pl.program_id(0)
    x = x_ref[...]
    r = x.shape[0]
    key, is_diag0 = _make_key(x, i, r)
    key = jnp.where(is_diag0, INT32_MIN, key)

    h16 = jax.lax.shift_right_arithmetic(key, 16).astype(jnp.int16)
    ls = key.astype(jnp.int16) ^ np.int16(-32768)
    del key

    def fold_count(v16):
        w = N
        while w > 32:
            w //= 2
            v16 = v16[:, :w] + v16[:, w:]
        return jnp.sum(v16.astype(jnp.int32), axis=1, keepdims=True)

    def stepA(carry, _):
        lo, hi, chi = carry
        mid = lo + jax.lax.shift_right_logical(hi - lo, 1)
        cnt = fold_count(jnp.where(h16 >= mid.astype(jnp.int16), ONE16, ZERO16))
        ge = cnt >= KOFF
        return (jnp.where(ge, mid, lo), jnp.where(ge, hi, mid),
                jnp.where(ge, chi, cnt)), None

    loA = jnp.full((r, 1), np.int32(-32768))
    hiA = jnp.full((r, 1), np.int32(32768))
    chiA = jnp.zeros((r, 1), jnp.int32)
    (hstar, _, cgt), _ = jax.lax.scan(stepA, (loA, hiA, chiA), None,
                                      length=16, unroll=16)
    rstar = KOFF - cgt

    inb = jnp.where(h16 == hstar.astype(jnp.int16), ONE16, ZERO16)

    def stepB(carry, _):
        lo, hi = carry
        mid = lo + jax.lax.shift_right_logical(hi - lo, 1)
        cnt = fold_count(jnp.where(ls >= mid.astype(jnp.int16), inb, ZERO16))
        ge = cnt >= rstar
        return (jnp.where(ge, mid, lo), jnp.where(ge, hi, mid)), None

    loB = jnp.full((r, 1), np.int32(-32768))
    hiB = jnp.full((r, 1), np.int32(32768))
    (lstar, _), _ = jax.lax.scan(stepB, (loB, hiB), None, length=16, unroll=16)

    thr = (jax.lax.shift_left(hstar, 16)
           | ((lstar ^ np.int32(0x8000)) & np.int32(0xFFFF)))

    key2, is_diag = _make_key(x, i, r)
    a = jax.nn.sigmoid(x)
    keep = jnp.logical_or(key2 >= thr, is_diag)
    a = jnp.where(is_diag, jnp.float32(1.0), a)
    o_ref[...] = jnp.where(keep, a, jnp.float32(0.0))


@jax.jit
def kernel(logits):
    grid = (N // BLOCK_ROWS,)
    return pl.pallas_call(
        _block_kernel,
        grid=grid,
        in_specs=[pl.BlockSpec((BLOCK_ROWS, N), lambda i: (i, 0))],
        out_specs=pl.BlockSpec((BLOCK_ROWS, N), lambda i: (i, 0)),
        out_shape=jax.ShapeDtypeStruct((N, N), jnp.float32),
        compiler_params=pltpu.CompilerParams(
            dimension_semantics=("arbitrary",),
        ),
    )(logits)


# speedup vs baseline: 1.1061x; 1.1061x over previous
import jax
import jax.numpy as jnp
import numpy as np
from jax.experimental import pallas as pl
from jax.experimental.pallas import tpu as pltpu

N = 4096
KOFF = 63
BLOCK_ROWS = 512
INT32_MIN = np.int32(-2147483648)
SIGN_LO = np.int32(0x7FFFFFFF)
ONE16 = np.int16(1)
ZERO16 = np.int16(0)


def _make_key(x, i, r):
    bits = jax.lax.bitcast_convert_type(x, jnp.int32)
    key = bits ^ (jax.lax.shift_right_arithmetic(bits, 31) & SIGN_LO)
    row = jax.lax.broadcasted_iota(jnp.int32, (r, N), 0) + i * r
    col = jax.lax.broadcasted_iota(jnp.int32, (r, N), 1)
    is_diag = row == col
    return jnp.where(is_diag, INT32_MIN, key), is_diag


def _block_kernel(x_ref, o_ref):
    i = pl.program_id(0)
    x = x_ref[...]
    r = x.shape[0]
    key, _ = _make_key(x, i, r)

    h16 = jax.lax.shift_right_arithmetic(key, 16).astype(jnp.int16)
    ls = key.astype(jnp.int16) ^ np.int16(-32768)
    del key

    def fold_count(v16):
        w = N
        while w > 128:
            w //= 2
            v16 = v16[:, :w] + v16[:, w:]
        return jnp.sum(v16.astype(jnp.int32), axis=1, keepdims=True)

    def stepA(carry, _):
        lo, hi, chi = carry
        mid = lo + jax.lax.shift_right_logical(hi - lo, 1)
        cnt = fold_count(jnp.where(h16 >= mid.astype(jnp.int16), ONE16, ZERO16))
        ge = cnt >= KOFF
        return (jnp.where(ge, mid, lo), jnp.where(ge, hi, mid),
                jnp.where(ge, chi, cnt)), None

    loA = jnp.full((r, 1), np.int32(-32768))
    hiA = jnp.full((r, 1), np.int32(32768))
    chiA = jnp.zeros((r, 1), jnp.int32)
    (hstar, _, cgt), _ = jax.lax.scan(stepA, (loA, hiA, chiA), None,
                                      length=16, unroll=16)
    rstar = KOFF - cgt

    inb = jnp.where(h16 == hstar.astype(jnp.int16), ONE16, ZERO16)

    def stepB(carry, _):
        lo, hi = carry
        mid = lo + jax.lax.shift_right_logical(hi - lo, 1)
        cnt = fold_count(jnp.where(ls >= mid.astype(jnp.int16), inb, ZERO16))
        ge = cnt >= rstar
        return (jnp.where(ge, mid, lo), jnp.where(ge, hi, mid)), None

    loB = jnp.full((r, 1), np.int32(-32768))
    hiB = jnp.full((r, 1), np.int32(32768))
    (lstar, _), _ = jax.lax.scan(stepB, (loB, hiB), None, length=16, unroll=16)

    thr = (jax.lax.shift_left(hstar, 16)
           | ((lstar ^ np.int32(0x8000)) & np.int32(0xFFFF)))

    key2, is_diag = _make_key(x, i, r)
    a = jax.nn.sigmoid(x)
    keep = jnp.logical_or(key2 >= thr, is_diag)
    a = jnp.where(is_diag, jnp.float32(1.0), a)
    o_ref[...] = jnp.where(keep, a, jnp.float32(0.0))


@jax.jit
def kernel(logits):
    grid = (N // BLOCK_ROWS,)
    return pl.pallas_call(
        _block_kernel,
        grid=grid,
        in_specs=[pl.BlockSpec((BLOCK_ROWS, N), lambda i: (i, 0))],
        out_specs=pl.BlockSpec((BLOCK_ROWS, N), lambda i: (i, 0)),
        out_shape=jax.ShapeDtypeStruct((N, N), jnp.float32),
        compiler_params=pltpu.CompilerParams(
            dimension_semantics=("arbitrary",),
        ),
    )(logits)
